# baseline (speedup 1.0000x reference)
import functools

import jax
import jax.numpy as jnp
from jax import lax
from jax.experimental import pallas as pl
from jax.experimental.pallas import tpu as pltpu
from jax.experimental.pallas import tpu_sc as plsc

NC = 2
NS = 16
NW = NC * NS
L = 16

BATCH = 16384
EMB = 64
NROWS = 1_000_000
BLOCKS = (NROWS + 127) // 128
BPW = (BLOCKS + NW - 1) // NW
SB = 4
SPW = (BPW + SB - 1) // SB
BPW_LOOP = SPW * SB
MAXBLK = (NW - 1) * BPW + BPW_LOOP
PAD_COLS = ((NROWS + 127) // 128) * 128
SUPER_CLAMP = PAD_COLS - SB * 128
NSTARTS = ((MAXBLK + 1 + L) + 15) // 16 * 16
ROWS_PER_W = BATCH // NW
PASS_ROWS = 128
NPASS = ROWS_PER_W // PASS_ROWS
RING = 256

_SENTINEL = jnp.iinfo(jnp.int32).max


def _sc_sweep():
    mesh = plsc.VectorSubcoreMesh(core_axis_name="c", subcore_axis_name="s")

    @functools.partial(
        pl.kernel,
        mesh=mesh,
        compiler_params=pltpu.CompilerParams(needs_layout_passes=False),
        out_type=[
            jax.ShapeDtypeStruct((BATCH * EMB,), jnp.float32),
        ],
        scratch_types=[
            pltpu.VMEM((BATCH + L,), jnp.int32),
            pltpu.VMEM((NSTARTS,), jnp.int32),
            pltpu.VMEM((EMB, SB * 128), jnp.float32),
            pltpu.VMEM((EMB, SB * 128), jnp.float32),
            pltpu.VMEM((RING * EMB,), jnp.float32),
            pltpu.SemaphoreType.DMA,
            pltpu.SemaphoreType.DMA,
        ],
    )
    def k(s_hbm, st_hbm, tab, stage,
          sidx_v, starts_v, blk0, blk1, ring_v, sem0, sem1):
        w = lax.axis_index("s") * NC + lax.axis_index("c")
        base_blk = w * BPW
        iota16 = lax.broadcasted_iota(jnp.int32, (L,), 0)

        if True:
            pltpu.sync_copy(s_hbm, sidx_v)
            pltpu.sync_copy(st_hbm, starts_v)
            s_w = starts_v[pl.ds(base_blk, L)][0]

            def sstart(ls):
                return jnp.minimum((base_blk + SB * ls) * 128,
                                   jnp.int32(SUPER_CLAMP))

            pltpu.async_copy(
                tab.at[:, pl.ds(sstart(0), SB * 128)], blk0, sem0)
            pltpu.async_copy(
                tab.at[:, pl.ds(sstart(1), SB * 128)], blk1, sem1)

            def process(blk, bufref, sbase):
                s_t = starts_v[pl.ds(blk, L)][0]
                e_t = starts_v[pl.ds(blk + 1, L)][0]
                for g in range(128 // L):
                    @pl.when(s_t + g * L < e_t)
                    def _(g=g, s_t=s_t, e_t=e_t, sbase=sbase):
                        kv = s_t + g * L + iota16
                        msk = kv < e_t
                        svals = plsc.load_gather(sidx_v, [kv], mask=msk)
                        cols = jnp.where(msk, svals - sbase, 0)
                        ringpos = ((kv - s_w) & (RING - 1)) * EMB
                        for c in range(EMB):
                            vals = plsc.load_gather(
                                bufref,
                                [jnp.full((L,), c, jnp.int32), cols],
                                mask=msk)
                            plsc.store_scatter(
                                ring_v, [ringpos + c], vals, mask=msk)
                return e_t

            def maybe_flush(end_rank, nfl):
                pend = end_rank - s_w - nfl

                @pl.when(pend >= 128)
                def _():
                    off = (nfl & (RING - 1)) * EMB
                    pltpu.sync_copy(
                        ring_v.at[pl.ds(off, 128 * EMB)],
                        stage.at[pl.ds((s_w + nfl) * EMB, 128 * EMB)])

                return jnp.where(pend >= 128, nfl + 128, nfl)

            def half(t2, par, buf, sem, nfl):
                ls = 2 * t2 + par
                sbase = sstart(ls)
                pltpu.make_async_copy(
                    tab.at[:, pl.ds(0, SB * 128)], buf, sem).wait()

                def subbody(sub, nfl, buf=buf, ls=ls, sbase=sbase):
                    blk = base_blk + SB * ls + sub
                    e_t = process(blk, buf, sbase)
                    return maybe_flush(e_t, nfl)

                nfl = lax.fori_loop(0, SB, subbody, nfl)
                pltpu.async_copy(
                    tab.at[:, pl.ds(sstart(ls + 2), SB * 128)], buf, sem)
                return nfl

            def body(t2, nfl):
                nfl = half(t2, 0, blk0, sem0, nfl)
                nfl = half(t2, 1, blk1, sem1, nfl)
                return nfl

            nfl = lax.fori_loop(0, SPW // 2, body, jnp.int32(0))

            pltpu.make_async_copy(
                tab.at[:, pl.ds(0, SB * 128)], blk0, sem0).wait()
            pltpu.make_async_copy(
                tab.at[:, pl.ds(0, SB * 128)], blk1, sem1).wait()

            e_last = starts_v[pl.ds(base_blk + BPW_LOOP, L)][0]
            pend = e_last - s_w - nfl
            off = jnp.int32(0)
            for sz in (128, 64, 32, 16, 8, 4, 2, 1):
                bit = (pend & sz) != 0
                cur_off = off

                @pl.when(bit)
                def _(sz=sz, cur_off=cur_off, nfl=nfl, stage=stage):
                    roff = ((nfl + cur_off) & (RING - 1)) * EMB
                    pltpu.sync_copy(
                        ring_v.at[pl.ds(roff, sz * EMB)],
                        stage.at[pl.ds((s_w + nfl + cur_off) * EMB, sz * EMB)])

                off = jnp.where(bit, off + sz, off)

    return k


def _sc_dot():
    mesh = plsc.VectorSubcoreMesh(core_axis_name="c", subcore_axis_name="s")

    @functools.partial(
        pl.kernel,
        mesh=mesh,
        out_type=[
            jax.ShapeDtypeStruct((NW * L,), jnp.float32),
            jax.ShapeDtypeStruct((BATCH,), jnp.float32),
        ],
        scratch_types=[
            pltpu.VMEM((ROWS_PER_W,), jnp.int32),
            pltpu.VMEM((ROWS_PER_W,), jnp.int32),
            pltpu.VMEM((ROWS_PER_W,), jnp.int32),
            pltpu.VMEM((ROWS_PER_W,), jnp.int32),
            pltpu.VMEM((PASS_ROWS * EMB,), jnp.float32),
            pltpu.VMEM((PASS_ROWS * EMB,), jnp.float32),
            pltpu.VMEM((PASS_ROWS * EMB,), jnp.float32),
            pltpu.VMEM((PASS_ROWS * EMB,), jnp.float32),
            pltpu.VMEM((ROWS_PER_W,), jnp.float32),
            pltpu.VMEM((ROWS_PER_W,), jnp.float32),
            pltpu.VMEM((ROWS_PER_W,), jnp.float32),
            pltpu.VMEM((L,), jnp.float32),
            pltpu.SemaphoreType.DMA,
            pltpu.SemaphoreType.DMA,
            pltpu.SemaphoreType.DMA,
            pltpu.SemaphoreType.DMA,
        ],
    )
    def k(invu_hbm, invb_hbm, stu_hbm, stb_hbm, uidx_hbm, bidx_hbm,
          ubias_hbm, bbias_hbm,
          partials_hbm, bsum_hbm,
          invu_v, invb_v, uidx_v, bidx_v, ur0, ur1, br0, br1,
          ubv_v, bbv_v, bsum_v, acc_v,
          sem_r0, sem_r1, sem_bu, sem_bb):
        wid = lax.axis_index("s") * NC + lax.axis_index("c")
        base = wid * ROWS_PER_W

        ubufs = (ur0, ur1)
        bbufs = (br0, br1)
        sems = (sem_r0, sem_r1)

        pltpu.sync_copy(invu_hbm.at[pl.ds(base, ROWS_PER_W)], invu_v)
        pltpu.sync_copy(invb_hbm.at[pl.ds(base, ROWS_PER_W)], invb_v)
        pltpu.sync_copy(uidx_hbm.at[pl.ds(base, ROWS_PER_W)], uidx_v)
        pltpu.sync_copy(bidx_hbm.at[pl.ds(base, ROWS_PER_W)], bidx_v)

        cp_bu = pltpu.async_copy(ubias_hbm.at[uidx_v], ubv_v, sem_bu)
        cp_bb = pltpu.async_copy(bbias_hbm.at[bidx_v], bbv_v, sem_bb)

        def enqueue_pass(p, ubuf, bbuf, sem):
            def enq(g, _):
                uvec = invu_v[pl.ds(p * PASS_ROWS + g * L, L)]
                bvec = invb_v[pl.ds(p * PASS_ROWS + g * L, L)]
                for j in range(L):
                    iu = uvec[j]
                    pltpu.async_copy(
                        stu_hbm.at[pl.ds(iu * EMB, EMB)],
                        ubuf.at[pl.ds((g * L + j) * EMB, EMB)], sem)
                    ib = bvec[j]
                    pltpu.async_copy(
                        stb_hbm.at[pl.ds(ib * EMB, EMB)],
                        bbuf.at[pl.ds((g * L + j) * EMB, EMB)], sem)
                return 0

            lax.fori_loop(0, PASS_ROWS // L, enq, 0)

        def drain_pass(ubuf, sem):
            def dr(g, _):
                for _j in range(2 * L):
                    pltpu.make_async_copy(
                        stu_hbm.at[pl.ds(0, EMB)],
                        ubuf.at[pl.ds(0, EMB)], sem).wait()
                return 0

            lax.fori_loop(0, PASS_ROWS // L, dr, 0)

        def compute_pass(ubuf, bbuf, accs):
            def body(rr, accs):
                a0, a1, a2, a3 = accs
                o = rr * EMB
                a0 = a0 + ubuf[pl.ds(o, L)] * bbuf[pl.ds(o, L)]
                a1 = a1 + ubuf[pl.ds(o + L, L)] * bbuf[pl.ds(o + L, L)]
                a2 = a2 + ubuf[pl.ds(o + 2 * L, L)] * bbuf[pl.ds(o + 2 * L, L)]
                a3 = a3 + ubuf[pl.ds(o + 3 * L, L)] * bbuf[pl.ds(o + 3 * L, L)]
                return a0, a1, a2, a3

            return lax.fori_loop(0, PASS_ROWS, body, accs)

        enqueue_pass(0, ubufs[0], bbufs[0], sems[0])

        zero = jnp.zeros((L,), jnp.float32)
        accs = (zero, zero, zero, zero)
        for p in range(NPASS):
            if p + 1 < NPASS:
                enqueue_pass(p + 1, ubufs[(p + 1) % 2], bbufs[(p + 1) % 2],
                             sems[(p + 1) % 2])
            drain_pass(ubufs[p % 2], sems[p % 2])
            accs = compute_pass(ubufs[p % 2], bbufs[p % 2], accs)

        cp_bu.wait()
        cp_bb.wait()
        for kk in range(ROWS_PER_W // L):
            s = pl.ds(kk * L, L)
            bsum_v[s] = ubv_v[s] + bbv_v[s]
        pltpu.sync_copy(bsum_v, bsum_hbm.at[pl.ds(base, ROWS_PER_W)])

        acc_v[...] = (accs[0] + accs[1]) + (accs[2] + accs[3])
        pltpu.sync_copy(acc_v, partials_hbm.at[pl.ds(wid * L, L)])

    return k


def _tc_finalize(partials, bias_sum):
    def body(p_ref, b_ref, o_ref):
        s = jnp.sum(p_ref[...])
        o_ref[...] = jax.nn.sigmoid(b_ref[...] + s)

    return pl.pallas_call(
        body,
        out_shape=jax.ShapeDtypeStruct(bias_sum.shape, jnp.float32),
    )(partials, bias_sum)


def _prep(idx):
    arange = jnp.arange(BATCH, dtype=jnp.int32)
    srt, perm = jax.lax.sort_key_val(idx, arange)
    first = jnp.concatenate(
        [jnp.ones((1,), jnp.bool_), srt[1:] != srt[:-1]])
    rank = (jnp.cumsum(first) - 1).astype(jnp.int32)
    _, inv = jax.lax.sort_key_val(perm, rank)
    uniq = jnp.where(first, srt, _SENTINEL)
    sdd16 = jnp.sort(uniq)
    sdd = jnp.concatenate([sdd16, jnp.full((L,), _SENTINEL, jnp.int32)])
    clamped = jnp.minimum(sdd16, jnp.int32(1_100_000))
    q = jnp.arange(MAXBLK + 1, dtype=jnp.int32) * 256
    keyed = jnp.concatenate([clamped * 2 + 1, q])
    tags = jnp.concatenate([jnp.full((BATCH,), -1, jnp.int32),
                            jnp.arange(MAXBLK + 1, dtype=jnp.int32)])
    _, tag = jax.lax.sort_key_val(keyed, tags)
    isq = tag >= 0
    creals = jnp.cumsum((~isq).astype(jnp.int32))
    ckey = jnp.where(isq, tag, jnp.int32(2**30))
    _, sv = jax.lax.sort_key_val(ckey, creals)
    starts = sv[:MAXBLK + 1]
    starts_pad = jnp.zeros((NSTARTS,), jnp.int32).at[:MAXBLK + 1].set(starts)
    return sdd, inv, starts_pad


def kernel(inputs, user_embedding, user_bias, book_embedding, book_bias):
    uidx = inputs[:, 0]
    bidx = inputs[:, 1]
    ub = user_bias.reshape(-1)
    bb = book_bias.reshape(-1)

    su, inv_u, starts_u = _prep(uidx)
    sb, inv_b, starts_b = _prep(bidx)

    sweep = _sc_sweep()
    stage_u, = sweep(su, starts_u, user_embedding.T)
    stage_b, = sweep(sb, starts_b, book_embedding.T)
    partials, bsum = _sc_dot()(
        inv_u, inv_b, stage_u, stage_b, uidx, bidx, ub, bb)
    out = _tc_finalize(partials.reshape(NW, L),
                       bsum.reshape(BATCH // 128, 128))
    return out.reshape(BATCH, 1)


# speedup vs baseline: 1.0713x; 1.0713x over previous
import functools

import jax
import jax.numpy as jnp
from jax import lax
from jax.experimental import pallas as pl
from jax.experimental.pallas import tpu as pltpu
from jax.experimental.pallas import tpu_sc as plsc

NC = 2
NS = 16
NW = NC * NS
L = 16

BATCH = 16384
EMB = 64
NROWS = 1_000_000
BLOCKS = (NROWS + 127) // 128
BPW = (BLOCKS + NW - 1) // NW
SB = 4
SPW = (BPW + SB - 1) // SB
BPW_LOOP = SPW * SB
MAXBLK = (NW - 1) * BPW + BPW_LOOP
PAD_COLS = ((NROWS + 127) // 128) * 128
SUPER_CLAMP = PAD_COLS - SB * 128
NSTARTS = ((MAXBLK + 1 + L) + 15) // 16 * 16
ROWS_PER_W = BATCH // NW
PASS_ROWS = 128
NPASS = ROWS_PER_W // PASS_ROWS
RING = 256

_SENTINEL = jnp.iinfo(jnp.int32).max


def _sc_sweep():
    mesh = plsc.VectorSubcoreMesh(core_axis_name="c", subcore_axis_name="s")

    @functools.partial(
        pl.kernel,
        mesh=mesh,
        compiler_params=pltpu.CompilerParams(needs_layout_passes=False),
        out_type=[
            jax.ShapeDtypeStruct((BATCH * EMB,), jnp.float32),
            jax.ShapeDtypeStruct((BATCH * EMB,), jnp.float32),
        ],
        scratch_types=[
            pltpu.VMEM((BATCH + L,), jnp.int32),
            pltpu.VMEM((NSTARTS,), jnp.int32),
            pltpu.VMEM((EMB, SB * 128), jnp.float32),
            pltpu.VMEM((EMB, SB * 128), jnp.float32),
            pltpu.VMEM((RING * EMB,), jnp.float32),
            pltpu.SemaphoreType.DMA,
            pltpu.SemaphoreType.DMA,
        ],
    )
    def k(su_hbm, sb_hbm, stu_hbm, stb_hbm_in, utT_hbm, btT_hbm,
          stu_out, stb_out,
          sidx_v, starts_v, blk0, blk1, ring_v, sem0, sem1):
        w = lax.axis_index("s") * NC + lax.axis_index("c")
        base_blk = w * BPW
        iota16 = lax.broadcasted_iota(jnp.int32, (L,), 0)

        for s_hbm, st_hbm, tab, stage in (
                (su_hbm, stu_hbm, utT_hbm, stu_out),
                (sb_hbm, stb_hbm_in, btT_hbm, stb_out)):
            pltpu.sync_copy(s_hbm, sidx_v)
            pltpu.sync_copy(st_hbm, starts_v)
            s_w = starts_v[pl.ds(base_blk, L)][0]

            def sstart(ls):
                return jnp.minimum((base_blk + SB * ls) * 128,
                                   jnp.int32(SUPER_CLAMP))

            pltpu.async_copy(
                tab.at[:, pl.ds(sstart(0), SB * 128)], blk0, sem0)
            pltpu.async_copy(
                tab.at[:, pl.ds(sstart(1), SB * 128)], blk1, sem1)

            def process(blk, bufref, sbase):
                s_t = starts_v[pl.ds(blk, L)][0]
                e_t = starts_v[pl.ds(blk + 1, L)][0]
                for g in range(128 // L):
                    @pl.when(s_t + g * L < e_t)
                    def _(g=g, s_t=s_t, e_t=e_t, sbase=sbase):
                        kv = s_t + g * L + iota16
                        msk = kv < e_t
                        svals = plsc.load_gather(sidx_v, [kv], mask=msk)
                        cols = jnp.where(msk, svals - sbase, 0)
                        ringpos = ((kv - s_w) & (RING - 1)) * EMB
                        for c in range(EMB):
                            vals = plsc.load_gather(
                                bufref,
                                [jnp.full((L,), c, jnp.int32), cols],
                                mask=msk)
                            plsc.store_scatter(
                                ring_v, [ringpos + c], vals, mask=msk)
                return e_t

            def maybe_flush(end_rank, nfl):
                pend = end_rank - s_w - nfl

                @pl.when(pend >= 128)
                def _():
                    off = (nfl & (RING - 1)) * EMB
                    pltpu.sync_copy(
                        ring_v.at[pl.ds(off, 128 * EMB)],
                        stage.at[pl.ds((s_w + nfl) * EMB, 128 * EMB)])

                return jnp.where(pend >= 128, nfl + 128, nfl)

            def half(t2, par, buf, sem, nfl):
                ls = 2 * t2 + par
                sbase = sstart(ls)
                pltpu.make_async_copy(
                    tab.at[:, pl.ds(0, SB * 128)], buf, sem).wait()

                def subbody(sub, nfl, buf=buf, ls=ls, sbase=sbase):
                    blk = base_blk + SB * ls + sub
                    e_t = process(blk, buf, sbase)
                    return maybe_flush(e_t, nfl)

                nfl = lax.fori_loop(0, SB, subbody, nfl)
                pltpu.async_copy(
                    tab.at[:, pl.ds(sstart(ls + 2), SB * 128)], buf, sem)
                return nfl

            def body(t2, nfl):
                nfl = half(t2, 0, blk0, sem0, nfl)
                nfl = half(t2, 1, blk1, sem1, nfl)
                return nfl

            nfl = lax.fori_loop(0, SPW // 2, body, jnp.int32(0))

            pltpu.make_async_copy(
                tab.at[:, pl.ds(0, SB * 128)], blk0, sem0).wait()
            pltpu.make_async_copy(
                tab.at[:, pl.ds(0, SB * 128)], blk1, sem1).wait()

            e_last = starts_v[pl.ds(base_blk + BPW_LOOP, L)][0]
            pend = e_last - s_w - nfl
            off = jnp.int32(0)
            for sz in (128, 64, 32, 16, 8, 4, 2, 1):
                bit = (pend & sz) != 0
                cur_off = off

                @pl.when(bit)
                def _(sz=sz, cur_off=cur_off, nfl=nfl, stage=stage):
                    roff = ((nfl + cur_off) & (RING - 1)) * EMB
                    pltpu.sync_copy(
                        ring_v.at[pl.ds(roff, sz * EMB)],
                        stage.at[pl.ds((s_w + nfl + cur_off) * EMB, sz * EMB)])

                off = jnp.where(bit, off + sz, off)

    return k


def _sc_dot():
    mesh = plsc.VectorSubcoreMesh(core_axis_name="c", subcore_axis_name="s")

    @functools.partial(
        pl.kernel,
        mesh=mesh,
        out_type=[
            jax.ShapeDtypeStruct((NW * L,), jnp.float32),
            jax.ShapeDtypeStruct((BATCH,), jnp.float32),
        ],
        scratch_types=[
            pltpu.VMEM((ROWS_PER_W,), jnp.int32),
            pltpu.VMEM((ROWS_PER_W,), jnp.int32),
            pltpu.VMEM((ROWS_PER_W,), jnp.int32),
            pltpu.VMEM((ROWS_PER_W,), jnp.int32),
            pltpu.VMEM((PASS_ROWS * EMB,), jnp.float32),
            pltpu.VMEM((PASS_ROWS * EMB,), jnp.float32),
            pltpu.VMEM((PASS_ROWS * EMB,), jnp.float32),
            pltpu.VMEM((PASS_ROWS * EMB,), jnp.float32),
            pltpu.VMEM((ROWS_PER_W,), jnp.float32),
            pltpu.VMEM((ROWS_PER_W,), jnp.float32),
            pltpu.VMEM((ROWS_PER_W,), jnp.float32),
            pltpu.VMEM((L,), jnp.float32),
            pltpu.SemaphoreType.DMA,
            pltpu.SemaphoreType.DMA,
            pltpu.SemaphoreType.DMA,
            pltpu.SemaphoreType.DMA,
        ],
    )
    def k(invu_hbm, invb_hbm, stu_hbm, stb_hbm, uidx_hbm, bidx_hbm,
          ubias_hbm, bbias_hbm,
          partials_hbm, bsum_hbm,
          invu_v, invb_v, uidx_v, bidx_v, ur0, ur1, br0, br1,
          ubv_v, bbv_v, bsum_v, acc_v,
          sem_r0, sem_r1, sem_bu, sem_bb):
        wid = lax.axis_index("s") * NC + lax.axis_index("c")
        base = wid * ROWS_PER_W

        ubufs = (ur0, ur1)
        bbufs = (br0, br1)
        sems = (sem_r0, sem_r1)

        pltpu.sync_copy(invu_hbm.at[pl.ds(base, ROWS_PER_W)], invu_v)
        pltpu.sync_copy(invb_hbm.at[pl.ds(base, ROWS_PER_W)], invb_v)
        pltpu.sync_copy(uidx_hbm.at[pl.ds(base, ROWS_PER_W)], uidx_v)
        pltpu.sync_copy(bidx_hbm.at[pl.ds(base, ROWS_PER_W)], bidx_v)

        cp_bu = pltpu.async_copy(ubias_hbm.at[uidx_v], ubv_v, sem_bu)
        cp_bb = pltpu.async_copy(bbias_hbm.at[bidx_v], bbv_v, sem_bb)

        def enqueue_pass(p, ubuf, bbuf, sem):
            def enq(g, _):
                uvec = invu_v[pl.ds(p * PASS_ROWS + g * L, L)]
                bvec = invb_v[pl.ds(p * PASS_ROWS + g * L, L)]
                for j in range(L):
                    iu = uvec[j]
                    pltpu.async_copy(
                        stu_hbm.at[pl.ds(iu * EMB, EMB)],
                        ubuf.at[pl.ds((g * L + j) * EMB, EMB)], sem)
                    ib = bvec[j]
                    pltpu.async_copy(
                        stb_hbm.at[pl.ds(ib * EMB, EMB)],
                        bbuf.at[pl.ds((g * L + j) * EMB, EMB)], sem)
                return 0

            lax.fori_loop(0, PASS_ROWS // L, enq, 0)

        def drain_pass(ubuf, sem):
            def dr(g, _):
                for _j in range(2 * L):
                    pltpu.make_async_copy(
                        stu_hbm.at[pl.ds(0, EMB)],
                        ubuf.at[pl.ds(0, EMB)], sem).wait()
                return 0

            lax.fori_loop(0, PASS_ROWS // L, dr, 0)

        def compute_pass(ubuf, bbuf, accs):
            def body(rr, accs):
                a0, a1, a2, a3 = accs
                o = rr * EMB
                a0 = a0 + ubuf[pl.ds(o, L)] * bbuf[pl.ds(o, L)]
                a1 = a1 + ubuf[pl.ds(o + L, L)] * bbuf[pl.ds(o + L, L)]
                a2 = a2 + ubuf[pl.ds(o + 2 * L, L)] * bbuf[pl.ds(o + 2 * L, L)]
                a3 = a3 + ubuf[pl.ds(o + 3 * L, L)] * bbuf[pl.ds(o + 3 * L, L)]
                return a0, a1, a2, a3

            return lax.fori_loop(0, PASS_ROWS, body, accs)

        enqueue_pass(0, ubufs[0], bbufs[0], sems[0])

        zero = jnp.zeros((L,), jnp.float32)
        accs = (zero, zero, zero, zero)
        for p in range(NPASS):
            if p + 1 < NPASS:
                enqueue_pass(p + 1, ubufs[(p + 1) % 2], bbufs[(p + 1) % 2],
                             sems[(p + 1) % 2])
            drain_pass(ubufs[p % 2], sems[p % 2])
            accs = compute_pass(ubufs[p % 2], bbufs[p % 2], accs)

        cp_bu.wait()
        cp_bb.wait()
        for kk in range(ROWS_PER_W // L):
            s = pl.ds(kk * L, L)
            bsum_v[s] = ubv_v[s] + bbv_v[s]
        pltpu.sync_copy(bsum_v, bsum_hbm.at[pl.ds(base, ROWS_PER_W)])

        acc_v[...] = (accs[0] + accs[1]) + (accs[2] + accs[3])
        pltpu.sync_copy(acc_v, partials_hbm.at[pl.ds(wid * L, L)])

    return k


def _tc_finalize(partials, bias_sum):
    def body(p_ref, b_ref, o_ref):
        s = jnp.sum(p_ref[...])
        o_ref[...] = jax.nn.sigmoid(b_ref[...] + s)

    return pl.pallas_call(
        body,
        out_shape=jax.ShapeDtypeStruct(bias_sum.shape, jnp.float32),
    )(partials, bias_sum)


def _prep(idx):
    arange = jnp.arange(BATCH, dtype=jnp.int32)
    srt, perm = jax.lax.sort_key_val(idx, arange)
    first = jnp.concatenate(
        [jnp.ones((1,), jnp.bool_), srt[1:] != srt[:-1]])
    rank = (jnp.cumsum(first) - 1).astype(jnp.int32)
    _, inv = jax.lax.sort_key_val(perm, rank)
    uniq = jnp.where(first, srt, _SENTINEL)
    sdd16 = jnp.sort(uniq)
    sdd = jnp.concatenate([sdd16, jnp.full((L,), _SENTINEL, jnp.int32)])
    clamped = jnp.minimum(sdd16, jnp.int32(1_100_000))
    q = jnp.arange(MAXBLK + 1, dtype=jnp.int32) * 256
    keyed = jnp.concatenate([clamped * 2 + 1, q])
    tags = jnp.concatenate([jnp.full((BATCH,), -1, jnp.int32),
                            jnp.arange(MAXBLK + 1, dtype=jnp.int32)])
    _, tag = jax.lax.sort_key_val(keyed, tags)
    isq = tag >= 0
    creals = jnp.cumsum((~isq).astype(jnp.int32))
    ckey = jnp.where(isq, tag, jnp.int32(2**30))
    _, sv = jax.lax.sort_key_val(ckey, creals)
    starts = sv[:MAXBLK + 1]
    starts_pad = jnp.zeros((NSTARTS,), jnp.int32).at[:MAXBLK + 1].set(starts)
    return sdd, inv, starts_pad


def kernel(inputs, user_embedding, user_bias, book_embedding, book_bias):
    uidx = inputs[:, 0]
    bidx = inputs[:, 1]
    ub = user_bias.reshape(-1)
    bb = book_bias.reshape(-1)

    su, inv_u, starts_u = _prep(uidx)
    sb, inv_b, starts_b = _prep(bidx)

    stage_u, stage_b = _sc_sweep()(
        su, sb, starts_u, starts_b, user_embedding.T, book_embedding.T)
    partials, bsum = _sc_dot()(
        inv_u, inv_b, stage_u, stage_b, uidx, bidx, ub, bb)
    out = _tc_finalize(partials.reshape(NW, L),
                       bsum.reshape(BATCH // 128, 128))
    return out.reshape(BATCH, 1)
